# baseline (speedup 1.0000x reference)
import functools

import jax
import jax.numpy as jnp
from jax import lax
from jax.experimental import pallas as pl
from jax.experimental.pallas import tpu as pltpu
from jax.experimental.pallas import tpu_sc as plsc

R = 16
D = R * R
NC, NS = 2, 16
NW = NC * NS
CH = 128
BB = 512


def _permute_rows(src, dst, nrows):
    col_base = lax.iota(jnp.int32, R) * R
    lane = lax.iota(jnp.int32, R)

    def row(r, carry):
        rvec = jnp.full((R,), 0, jnp.int32) + r
        for i in range(R):
            v = plsc.load_gather(src, [rvec, col_base + i])
            plsc.store_scatter(dst, [rvec, lane + i * R], v)
        return carry

    lax.fori_loop(0, nrows, row, 0)


def _build_sc_gather_one(B, permute):
    b_per_w = B // NW
    nch = b_per_w // CH
    mesh = plsc.VectorSubcoreMesh(core_axis_name="c", subcore_axis_name="s")
    out_sds = jax.ShapeDtypeStruct((B, D), jnp.float32)

    @functools.partial(
        pl.kernel,
        out_type=out_sds,
        mesh=mesh,
        scratch_types=[
            pltpu.VMEM((b_per_w,), jnp.int32),
            pltpu.VMEM((CH, D), jnp.float32),
            pltpu.VMEM((CH, D), jnp.float32),
            pltpu.VMEM((CH, D), jnp.float32),
            pltpu.SemaphoreType.DMA,
            pltpu.SemaphoreType.DMA,
        ],
        compiler_params=pltpu.CompilerParams(
            use_tc_tiling_on_sc=True, needs_layout_passes=False),
    )
    def sc_gather(tab, idx_hbm, out, idx_v, buf_a, buf_b, buf_p, sem_a, sem_b):
        wid = lax.axis_index("s") * NC + lax.axis_index("c")
        base = wid * b_per_w
        pltpu.sync_copy(idx_hbm.at[pl.ds(base, b_per_w)], idx_v)

        bufs = (buf_a, buf_b)
        sems = (sem_a, sem_b)

        def issue(k):
            return pltpu.async_copy(
                tab.at[idx_v.at[pl.ds(k * CH, CH)]], bufs[k % 2], sems[k % 2])

        descs = [None] * nch
        descs[0] = issue(0)
        for k in range(nch):
            if k + 1 < nch:
                descs[k + 1] = issue(k + 1)
            descs[k].wait()
            src = bufs[k % 2]
            if permute:
                _permute_rows(src, buf_p, CH)
                src = buf_p
            pltpu.sync_copy(src, out.at[pl.ds(base + k * CH, CH)])

    return sc_gather


def _tc_body(g0_ref, g1_ref, g2_ref, s0_ref, s1_ref, out_ref):
    bb = g0_ref.shape[0]
    a0 = g0_ref[...] * s0_ref[0:1, :]
    a1 = g1_ref[...] * s1_ref[0:1, :]
    a2 = g2_ref[...]
    t0 = a0.T.reshape(R, R, bb)
    t1 = a1.T.reshape(R, R, bb)
    t2 = a2.T.reshape(R, R, bb)
    acc = [t0[:, j:j + 1, :] * t1[j:j + 1, :, :] for j in range(4)]
    for j in range(4, R):
        acc[j % 4] = acc[j % 4] + t0[:, j:j + 1, :] * t1[j:j + 1, :, :]
    m = (acc[0] + acc[1]) + (acc[2] + acc[3])
    out = jnp.sum(m * t2, axis=(0, 1))
    out_ref[...] = out.reshape(1, 1, bb)


def _tc_contract(g0, g1, g2p, s0, s1):
    B = g0.shape[0]
    nb = B // BB
    out = pl.pallas_call(
        _tc_body,
        grid=(nb,),
        in_specs=[
            pl.BlockSpec((BB, D), lambda i: (i, 0)),
            pl.BlockSpec((BB, D), lambda i: (i, 0)),
            pl.BlockSpec((BB, D), lambda i: (i, 0)),
            pl.BlockSpec((8, D), lambda i: (0, 0)),
            pl.BlockSpec((8, D), lambda i: (0, 0)),
        ],
        out_specs=pl.BlockSpec((1, 1, BB), lambda i: (i, 0, 0)),
        out_shape=jax.ShapeDtypeStruct((nb, 1, BB), jnp.float32),
    )(g0, g1, g2p, s0, s1)
    return out.reshape(B)


def kernel(core0, core1, core2, lam0, lam1, lam2, index):
    B = index.shape[0]
    dim = core0.shape[0]
    idx = index.astype(jnp.int32)
    i0, i1, i2 = idx[:, 0], idx[:, 1], idx[:, 2]
    c0 = core0.reshape(dim, D)
    c1 = core1.reshape(core1.shape[0], D)
    c2 = jnp.transpose(core2, (0, 2, 1)).reshape(core2.shape[0], D)

    gather = _build_sc_gather_one(B, False)
    g0 = gather(c0, i0)
    g1 = gather(c1, i1)
    g2p = gather(c2, i2)

    s0 = jnp.broadcast_to((lam2[:, None] * lam0[None, :]).reshape(1, D), (8, D))
    s1 = jnp.broadcast_to(jnp.tile(lam1, R).reshape(1, D), (8, D))
    return _tc_contract(g0, g1, g2p, s0, s1)


# speedup vs baseline: 1.1414x; 1.1414x over previous
import functools

import jax
import jax.numpy as jnp
from jax import lax
from jax.experimental import pallas as pl
from jax.experimental.pallas import tpu as pltpu
from jax.experimental.pallas import tpu_sc as plsc

R = 16
D = R * R
NC, NS = 2, 16
NW = NC * NS
CH = 128
BB = 512


def _permute_rows(src, dst, nrows):
    col_base = lax.iota(jnp.int32, R) * R
    lane = lax.iota(jnp.int32, R)

    def row(r, carry):
        rvec = jnp.full((R,), 0, jnp.int32) + r
        for i in range(R):
            v = plsc.load_gather(src, [rvec, col_base + i])
            plsc.store_scatter(dst, [rvec, lane + i * R], v)
        return carry

    lax.fori_loop(0, nrows, row, 0)


def _build_sc_gather_one(B, permute):
    b_per_w = B // NW
    nch = b_per_w // CH
    mesh = plsc.VectorSubcoreMesh(core_axis_name="c", subcore_axis_name="s")
    out_sds = jax.ShapeDtypeStruct((B, D), jnp.float32)

    @functools.partial(
        pl.kernel,
        out_type=out_sds,
        mesh=mesh,
        scratch_types=[
            pltpu.VMEM((b_per_w,), jnp.int32),
            pltpu.VMEM((CH, D), jnp.float32),
            pltpu.VMEM((CH, D), jnp.float32),
            pltpu.VMEM((CH, D), jnp.float32),
            pltpu.SemaphoreType.DMA,
            pltpu.SemaphoreType.DMA,
        ],
        compiler_params=pltpu.CompilerParams(
            use_tc_tiling_on_sc=True, needs_layout_passes=False),
    )
    def sc_gather(tab, idx_hbm, out, idx_v, buf_a, buf_b, buf_p, sem_a, sem_b):
        wid = lax.axis_index("s") * NC + lax.axis_index("c")
        base = wid * b_per_w
        pltpu.sync_copy(idx_hbm.at[pl.ds(base, b_per_w)], idx_v)

        bufs = (buf_a, buf_b)
        sems = (sem_a, sem_b)

        def issue(k):
            return pltpu.async_copy(
                tab.at[idx_v.at[pl.ds(k * CH, CH)]], bufs[k % 2], sems[k % 2])

        descs = [None] * nch
        descs[0] = issue(0)
        for k in range(nch):
            if k + 1 < nch:
                descs[k + 1] = issue(k + 1)
            descs[k].wait()
            src = bufs[k % 2]
            if permute:
                _permute_rows(src, buf_p, CH)
                src = buf_p
            pltpu.sync_copy(src, out.at[pl.ds(base + k * CH, CH)])

    return sc_gather


def _tc_body(g0_ref, g1_ref, g2_ref, s0_ref, s1_ref, out_ref):
    bb = g0_ref.shape[0]
    a0 = g0_ref[...] * s0_ref[0:1, :]
    a1 = g1_ref[...] * s1_ref[0:1, :]
    a2 = g2_ref[...]
    t0 = a0.T.reshape(R, R, bb)
    t1 = a1.T.reshape(R, R, bb)
    t2 = a2.T.reshape(R, R, bb)
    acc = [t0[:, j:j + 1, :] * t1[j:j + 1, :, :] for j in range(4)]
    for j in range(4, R):
        acc[j % 4] = acc[j % 4] + t0[:, j:j + 1, :] * t1[j:j + 1, :, :]
    m = (acc[0] + acc[1]) + (acc[2] + acc[3])
    out = jnp.sum(m * t2, axis=(0, 1))
    out_ref[...] = out.reshape(1, 1, bb)


def _tc_contract(g0, g1, g2p, s0, s1):
    B = g0.shape[0]
    nb = B // BB
    out = pl.pallas_call(
        _tc_body,
        grid=(nb,),
        in_specs=[
            pl.BlockSpec((BB, D), lambda i: (i, 0)),
            pl.BlockSpec((BB, D), lambda i: (i, 0)),
            pl.BlockSpec((BB, D), lambda i: (i, 0)),
            pl.BlockSpec((8, D), lambda i: (0, 0)),
            pl.BlockSpec((8, D), lambda i: (0, 0)),
        ],
        out_specs=pl.BlockSpec((1, 1, BB), lambda i: (i, 0, 0)),
        out_shape=jax.ShapeDtypeStruct((nb, 1, BB), jnp.float32),
    )(g0, g1, g2p, s0, s1)
    return out.reshape(B)


def kernel(core0, core1, core2, lam0, lam1, lam2, index):
    B = index.shape[0]
    dim = core0.shape[0]
    idx = index.astype(jnp.int32)
    i0, i1, i2 = idx[:, 0], idx[:, 1], idx[:, 2]
    c0 = core0.reshape(dim, D)
    c1 = core1.reshape(core1.shape[0], D)
    c2 = core2.reshape(core2.shape[0], D)

    g2p = _build_sc_gather_one(B, True)(c2, i2)
    gather = _build_sc_gather_one(B, False)
    g0 = gather(c0, i0)
    g1 = gather(c1, i1)

    s0 = jnp.broadcast_to((lam2[:, None] * lam0[None, :]).reshape(1, D), (8, D))
    s1 = jnp.broadcast_to(jnp.tile(lam1, R).reshape(1, D), (8, D))
    return _tc_contract(g0, g1, g2p, s0, s1)


# speedup vs baseline: 1.1591x; 1.0155x over previous
import functools

import jax
import jax.numpy as jnp
from jax import lax
from jax.experimental import pallas as pl
from jax.experimental.pallas import tpu as pltpu
from jax.experimental.pallas import tpu_sc as plsc

R = 16
D = R * R
NC, NS = 2, 16
NW = NC * NS
CH = 128
BB = 1024


def _permute_rows(src, dst, nrows):
    col_base = lax.iota(jnp.int32, R) * R
    lane = lax.iota(jnp.int32, R)

    def row(r, carry):
        rvec = jnp.full((R,), 0, jnp.int32) + r
        for i in range(R):
            v = plsc.load_gather(src, [rvec, col_base + i])
            plsc.store_scatter(dst, [rvec, lane + i * R], v)
        return carry

    lax.fori_loop(0, nrows, row, 0)


def _build_sc_gather_one(B, permute):
    b_per_w = B // NW
    nch = b_per_w // CH
    mesh = plsc.VectorSubcoreMesh(core_axis_name="c", subcore_axis_name="s")
    out_sds = jax.ShapeDtypeStruct((B, D), jnp.float32)

    @functools.partial(
        pl.kernel,
        out_type=out_sds,
        mesh=mesh,
        scratch_types=[
            pltpu.VMEM((b_per_w,), jnp.int32),
            pltpu.VMEM((CH, D), jnp.float32),
            pltpu.VMEM((CH, D), jnp.float32),
            pltpu.VMEM((CH, D), jnp.float32),
            pltpu.SemaphoreType.DMA,
            pltpu.SemaphoreType.DMA,
        ],
        compiler_params=pltpu.CompilerParams(
            use_tc_tiling_on_sc=True, needs_layout_passes=False),
    )
    def sc_gather(tab, idx_hbm, out, idx_v, buf_a, buf_b, buf_p, sem_a, sem_b):
        wid = lax.axis_index("s") * NC + lax.axis_index("c")
        base = wid * b_per_w
        pltpu.sync_copy(idx_hbm.at[pl.ds(base, b_per_w)], idx_v)

        bufs = (buf_a, buf_b)
        sems = (sem_a, sem_b)

        def issue(k):
            return pltpu.async_copy(
                tab.at[idx_v.at[pl.ds(k * CH, CH)]], bufs[k % 2], sems[k % 2])

        descs = [None] * nch
        descs[0] = issue(0)
        for k in range(nch):
            if k + 1 < nch:
                descs[k + 1] = issue(k + 1)
            descs[k].wait()
            src = bufs[k % 2]
            if permute:
                _permute_rows(src, buf_p, CH)
                src = buf_p
            pltpu.sync_copy(src, out.at[pl.ds(base + k * CH, CH)])

    return sc_gather


def _tc_body(g0_ref, g1_ref, g2_ref, s0_ref, s1_ref, out_ref):
    bb = g0_ref.shape[0]
    a0 = g0_ref[...] * s0_ref[0:1, :]
    a1 = g1_ref[...] * s1_ref[0:1, :]
    a2 = g2_ref[...]
    t0 = a0.T.reshape(R, R, bb)
    t1 = a1.T.reshape(R, R, bb)
    t2 = a2.T.reshape(R, R, bb)
    acc = [t0[:, j:j + 1, :] * t1[j:j + 1, :, :] for j in range(4)]
    for j in range(4, R):
        acc[j % 4] = acc[j % 4] + t0[:, j:j + 1, :] * t1[j:j + 1, :, :]
    m = (acc[0] + acc[1]) + (acc[2] + acc[3])
    out = jnp.sum(m * t2, axis=(0, 1))
    out_ref[...] = out.reshape(1, 1, bb)


def _tc_contract(g0, g1, g2p, s0, s1):
    B = g0.shape[0]
    nb = B // BB
    out = pl.pallas_call(
        _tc_body,
        grid=(nb,),
        in_specs=[
            pl.BlockSpec((BB, D), lambda i: (i, 0)),
            pl.BlockSpec((BB, D), lambda i: (i, 0)),
            pl.BlockSpec((BB, D), lambda i: (i, 0)),
            pl.BlockSpec((8, D), lambda i: (0, 0)),
            pl.BlockSpec((8, D), lambda i: (0, 0)),
        ],
        out_specs=pl.BlockSpec((1, 1, BB), lambda i: (i, 0, 0)),
        out_shape=jax.ShapeDtypeStruct((nb, 1, BB), jnp.float32),
    )(g0, g1, g2p, s0, s1)
    return out.reshape(B)


def kernel(core0, core1, core2, lam0, lam1, lam2, index):
    B = index.shape[0]
    dim = core0.shape[0]
    idx = index.astype(jnp.int32)
    i0, i1, i2 = idx[:, 0], idx[:, 1], idx[:, 2]
    c0 = core0.reshape(dim, D)
    c1 = core1.reshape(core1.shape[0], D)
    c2 = core2.reshape(core2.shape[0], D)

    g2p = _build_sc_gather_one(B, True)(c2, i2)
    gather = _build_sc_gather_one(B, False)
    g0 = gather(c0, i0)
    g1 = gather(c1, i1)

    s0 = jnp.broadcast_to((lam2[:, None] * lam0[None, :]).reshape(1, D), (8, D))
    s1 = jnp.broadcast_to(jnp.tile(lam1, R).reshape(1, D), (8, D))
    return _tc_contract(g0, g1, g2p, s0, s1)
